# baseline (speedup 1.0000x reference)
import functools

import jax
import jax.numpy as jnp
from jax import lax
from jax.experimental import pallas as pl
from jax.experimental.pallas import tpu as pltpu
from jax.experimental.pallas import tpu_sc as plsc

B = 256
K = 4096
H, W = 210, 160
HW = H * W
HP, WP = H + 2, W + 32
CPAD = HP * WP
HB = 30
NBLK = H // HB
HB2 = HB + 2
BW2 = HB2 * WP

NC, NS = 2, 16
NW = NC * NS
SPW = B // NW

_mesh = plsc.VectorSubcoreMesh(
    core_axis_name="c", subcore_axis_name="s", num_cores=NC, num_subcores=NS
)

BHALF = B // 2
SPWH = BHALF // NW


def _make_scatter(base):
    @functools.partial(
        pl.kernel,
        out_type=jax.ShapeDtypeStruct((NBLK * BHALF, BW2), jnp.float32),
        mesh=_mesh,
        compiler_params=pltpu.CompilerParams(needs_layout_passes=False),
        scratch_types=[
            pltpu.VMEM((K,), jnp.int32),
            pltpu.VMEM((K,), jnp.float32),
            pltpu.VMEM((K,), jnp.int32),
            pltpu.VMEM((K,), jnp.int32),
            pltpu.VMEM((CPAD,), jnp.float32),
            pltpu.VMEM((CPAD,), jnp.float32),
            pltpu.SemaphoreType.DMA,
            pltpu.SemaphoreType.DMA,
        ],
    )
    def _scatter_sc(
        idx_hbm, val_hbm, out_hbm, idx_v, val_v, pidx0, pidx1, can0, can1, sem0, sem1
    ):
        wid = lax.axis_index("s") * NC + lax.axis_index("c")
        zeros = jnp.zeros((16,), jnp.float32)
        canvases, pidxs, sems = (can0, can1), (pidx0, pidx1), (sem0, sem1)

        @pl.loop(0, CPAD // 16, unroll=8)
        def _zero0(i):
            can0[pl.ds(i * 16, 16)] = zeros

        @pl.loop(0, CPAD // 16, unroll=8)
        def _zero1(i):
            can1[pl.ds(i * 16, 16)] = zeros

        descs = [None, None]
        for j in range(SPWH):
            p = j % 2
            canvas_v, pidx_v, sem = canvases[p], pidxs[p], sems[p]
            sl = wid * SPWH + j
            pltpu.sync_copy(idx_hbm.at[base + sl, 0], idx_v)
            pltpu.sync_copy(val_hbm.at[base + sl, 0], val_v)

            if descs[p] is not None:
                for d in descs[p]:
                    d.wait()

                @pl.loop(0, K // 16, unroll=4)
                def _rezero(c):
                    pv = pidx_v[pl.ds(c * 16, 16)]
                    plsc.store_scatter(canvas_v, [pv], zeros)

            @pl.loop(0, K // 16, unroll=4)
            def _scat(c):
                iv = idx_v[pl.ds(c * 16, 16)]
                vv = val_v[pl.ds(c * 16, 16)]
                row = jnp.right_shift(iv * 26215, 22)
                pidx = iv + row * (WP - W) + (WP + 1)
                pidx_v[pl.ds(c * 16, 16)] = pidx
                plsc.store_scatter(canvas_v, [pidx], vv)

            descs[p] = [
                pltpu.async_copy(
                    canvas_v.at[pl.ds(i * HB * WP, BW2)],
                    out_hbm.at[i * BHALF + sl],
                    sem,
                )
                for i in range(NBLK)
            ]
        for p in range(2):
            if descs[p] is not None:
                for d in descs[p]:
                    d.wait()

    return _scatter_sc


_sc_h0 = _make_scatter(0)
_sc_h1 = _make_scatter(BHALF)


BH = 128


def _conv_compute(x_hbm, w_ref, b_ref, o_ref, scr_ref, bufs, sems):
    i = pl.program_id(0)

    def copy(blk):
        return pltpu.make_async_copy(
            x_hbm.at[pl.ds(blk * BH, BH)], bufs.at[blk % 2], sems.at[blk % 2]
        )

    @pl.when(i == 0)
    def _prologue():
        copy(0).start()

    @pl.when(i + 1 < NBLK)
    def _prefetch():
        copy(i + 1).start()

    copy(i).wait()
    x = bufs[i % 2]
    t = jnp.transpose(x)
    r = t.reshape(HB2, WP, BH)
    for dx in (1, 2):
        scr_ref[dx - 1] = r[:, dx : dx + W, :]
    for o in range(3):
        acc = None
        for dy in range(3):
            for dx in range(3):
                src = r[:, 0:W, :] if dx == 0 else scr_ref[dx - 1]
                v = w_ref[o, dy, dx] * src[dy : dy + HB]
                acc = v if acc is None else acc + v
        acc = acc + b_ref[o]
        o_ref[o] = 1.0 / (1.0 + jnp.exp(-acc))


def _make_conv(h, aliased):
    in_specs = [
        pl.BlockSpec(memory_space=pl.ANY),
        pl.BlockSpec(memory_space=pltpu.SMEM),
        pl.BlockSpec(memory_space=pltpu.SMEM),
    ]
    if aliased:
        in_specs.append(pl.BlockSpec(memory_space=pl.ANY))

        def body(x_ref, w_ref, b_ref, y_ref, o_ref, scr_ref, bufs, sems):
            del y_ref
            _conv_compute(x_ref, w_ref, b_ref, o_ref, scr_ref, bufs, sems)

    else:
        body = _conv_compute
    return pl.pallas_call(
        body,
        grid=(NBLK,),
        in_specs=in_specs,
        out_specs=pl.BlockSpec((3, HB, W, BH), lambda i: (0, i, 0, h)),
        out_shape=jax.ShapeDtypeStruct((3, H, W, B), jnp.float32),
        scratch_shapes=[
            pltpu.VMEM((2, HB2, W, BH), jnp.float32),
            pltpu.VMEM((2, BH, BW2), jnp.float32),
            pltpu.SemaphoreType.DMA((2,)),
        ],
        input_output_aliases={3: 0} if aliased else {},
    )


_conv_h0 = _make_conv(0, aliased=False)
_conv_h1 = _make_conv(1, aliased=True)


def kernel(top_k, idx, W_arr, b):
    w3 = W_arr.reshape(3, 3, 3)
    o1 = _sc_h0(idx, top_k)
    o2 = _sc_h1(idx, top_k)
    y1 = _conv_h0(o1, w3, b)
    y2 = _conv_h1(o2, w3, b, y1)
    return jnp.transpose(y2, (3, 0, 1, 2))


# speedup vs baseline: 1.0087x; 1.0087x over previous
import functools

import jax
import jax.numpy as jnp
from jax import lax
from jax.experimental import pallas as pl
from jax.experimental.pallas import tpu as pltpu
from jax.experimental.pallas import tpu_sc as plsc

B = 256
K = 4096
H, W = 210, 160
HW = H * W
HP, WP = H + 2, W + 32
CPAD = HP * WP
HB = 30
NBLK = H // HB
HB2 = HB + 2
BW2 = HB2 * WP

NC, NS = 2, 16
NW = NC * NS
SPW = B // NW

_mesh = plsc.VectorSubcoreMesh(
    core_axis_name="c", subcore_axis_name="s", num_cores=NC, num_subcores=NS
)

BHALF = B // 2
SPWH = BHALF // NW


def _make_scatter(base):
    @functools.partial(
        pl.kernel,
        out_type=jax.ShapeDtypeStruct((NBLK * BHALF, BW2), jnp.float32),
        mesh=_mesh,
        compiler_params=pltpu.CompilerParams(needs_layout_passes=False),
        scratch_types=[
            pltpu.VMEM((K,), jnp.int32),
            pltpu.VMEM((K,), jnp.float32),
            pltpu.VMEM((K,), jnp.int32),
            pltpu.VMEM((K,), jnp.int32),
            pltpu.VMEM((CPAD,), jnp.float32),
            pltpu.VMEM((CPAD,), jnp.float32),
            pltpu.SemaphoreType.DMA,
            pltpu.SemaphoreType.DMA,
        ],
    )
    def _scatter_sc(
        idx_hbm, val_hbm, out_hbm, idx_v, val_v, pidx0, pidx1, can0, can1, sem0, sem1
    ):
        wid = lax.axis_index("s") * NC + lax.axis_index("c")
        zeros = jnp.zeros((16,), jnp.float32)
        canvases, pidxs, sems = (can0, can1), (pidx0, pidx1), (sem0, sem1)

        @pl.loop(0, CPAD // 16, unroll=8)
        def _zero0(i):
            can0[pl.ds(i * 16, 16)] = zeros

        @pl.loop(0, CPAD // 16, unroll=8)
        def _zero1(i):
            can1[pl.ds(i * 16, 16)] = zeros

        descs = [None, None]
        for j in range(SPWH):
            p = j % 2
            canvas_v, pidx_v, sem = canvases[p], pidxs[p], sems[p]
            sl = wid * SPWH + j
            pltpu.sync_copy(idx_hbm.at[base + sl, 0], idx_v)
            pltpu.sync_copy(val_hbm.at[base + sl, 0], val_v)

            if descs[p] is not None:
                for d in descs[p]:
                    d.wait()

                @pl.loop(0, K // 16, unroll=4)
                def _rezero(c):
                    pv = pidx_v[pl.ds(c * 16, 16)]
                    plsc.store_scatter(canvas_v, [pv], zeros)

            @pl.loop(0, K // 16, unroll=4)
            def _scat(c):
                iv = idx_v[pl.ds(c * 16, 16)]
                vv = val_v[pl.ds(c * 16, 16)]
                row = jnp.right_shift(iv * 26215, 22)
                pidx = iv + row * (WP - W) + (WP + 1)
                pidx_v[pl.ds(c * 16, 16)] = pidx
                plsc.store_scatter(canvas_v, [pidx], vv)

            descs[p] = [
                pltpu.async_copy(
                    canvas_v.at[pl.ds(i * HB * WP, BW2)],
                    out_hbm.at[i * BHALF + sl],
                    sem,
                )
                for i in range(NBLK)
            ]
        for p in range(2):
            if descs[p] is not None:
                for d in descs[p]:
                    d.wait()

    return _scatter_sc


_sc_h0 = _make_scatter(0)
_sc_h1 = _make_scatter(BHALF)


BH = 128


def _conv_compute(x_ref, w_ref, b_ref, o_ref, scr_ref):
    x = x_ref[...]
    t = jnp.transpose(x)
    r = t.reshape(HB2, WP, BH)
    for dx in (1, 2):
        scr_ref[dx - 1] = r[:, dx : dx + W, :]
    for o in range(3):
        acc = None
        for dy in range(3):
            for dx in range(3):
                src = r[:, 0:W, :] if dx == 0 else scr_ref[dx - 1]
                v = w_ref[o, dy, dx] * src[dy : dy + HB]
                acc = v if acc is None else acc + v
        acc = acc + b_ref[o]
        o_ref[o] = 1.0 / (1.0 + jnp.exp(-acc))


def _make_conv(h, aliased):
    in_specs = [
        pl.BlockSpec((BH, BW2), lambda i: (i, 0)),
        pl.BlockSpec(memory_space=pltpu.SMEM),
        pl.BlockSpec(memory_space=pltpu.SMEM),
    ]
    if aliased:
        in_specs.append(pl.BlockSpec(memory_space=pl.ANY))

        def body(x_ref, w_ref, b_ref, y_ref, o_ref, scr_ref):
            del y_ref
            _conv_compute(x_ref, w_ref, b_ref, o_ref, scr_ref)

    else:
        body = _conv_compute
    return pl.pallas_call(
        body,
        grid=(NBLK,),
        in_specs=in_specs,
        out_specs=pl.BlockSpec((3, HB, W, BH), lambda i: (0, i, 0, h)),
        out_shape=jax.ShapeDtypeStruct((3, H, W, B), jnp.float32),
        scratch_shapes=[pltpu.VMEM((2, HB2, W, BH), jnp.float32)],
        input_output_aliases={3: 0} if aliased else {},
    )


_conv_h0 = _make_conv(0, aliased=False)
_conv_h1 = _make_conv(1, aliased=True)


def kernel(top_k, idx, W_arr, b):
    w3 = W_arr.reshape(3, 3, 3)
    o1 = _sc_h0(idx, top_k)
    o2 = _sc_h1(idx, top_k)
    y1 = _conv_h0(o1, w3, b)
    y2 = _conv_h1(o2, w3, b, y1)
    return jnp.transpose(y2, (3, 0, 1, 2))
